# Initial kernel scaffold; baseline (speedup 1.0000x reference)
import jax
import jax.numpy as jnp
from jax.experimental import pallas as pl


def kernel(y0, y1, y2, y3, y4, cb0_0, cb0_1, cb0_2, cb1_0, cb1_1, cb1_2, cb2_0, cb2_1, cb2_2, cb3_0, cb3_1, cb3_2, cb4_0, cb4_1, cb4_2):
    raise NotImplementedError("write your pallas kernel here")


# speedup vs baseline: 1.1437x; 1.1437x over previous
import jax
import jax.numpy as jnp
from jax.experimental import pallas as pl

_EPS = 1e-07
_KT = 2048


def _vq_group_body(x_ref, cb0_ref, cb1_ref, cb2_ref, q_ref, rate_ref):
    x0 = x_ref[0]
    T = x0.shape[0]
    r = x0
    rate = jnp.zeros((1, 1), jnp.float32)
    for cb_ref in (cb0_ref, cb1_ref, cb2_ref):
        K = cb_ref.shape[1]
        KT = min(K, _KT)
        nt = K // KT
        M = S = A = None
        for t in range(nt):
            cbt = cb_ref[0, t * KT:(t + 1) * KT, :]
            dots = jax.lax.dot_general(
                r, cbt, (((1,), (1,)), ((), ())),
                preferred_element_type=jnp.float32)
            cbn = jnp.sum(cbt * cbt, axis=1)
            tt = 2.0 * dots - cbn[None, :]
            tmax = jnp.max(tt, axis=1, keepdims=True)
            iota = jax.lax.broadcasted_iota(jnp.int32, (T, KT), 1) + t * KT
            targ = jnp.min(jnp.where(tt == tmax, iota, jnp.int32(K)),
                           axis=1, keepdims=True)
            tsum = jnp.sum(jnp.exp(tt - tmax), axis=1, keepdims=True)
            if t == 0:
                M, S, A = tmax, tsum, targ
            else:
                better = tmax > M
                Mn = jnp.maximum(M, tmax)
                S = S * jnp.exp(M - Mn) + tsum * jnp.exp(tmax - Mn)
                A = jnp.where(better, targ, A)
                M = Mn
        sel = None
        for t in range(nt):
            cbt = cb_ref[0, t * KT:(t + 1) * KT, :]
            iota = jax.lax.broadcasted_iota(jnp.int32, (T, KT), 1) + t * KT
            oh = (iota == A).astype(jnp.float32)
            g = jax.lax.dot_general(
                oh, cbt, (((1,), (0,)), ((), ())),
                precision=jax.lax.Precision.HIGHEST,
                preferred_element_type=jnp.float32)
            sel = g if sel is None else sel + g
        r = r - sel
        rate = rate + jnp.sum(-jnp.log(1.0 / S + _EPS), keepdims=True)
    q_ref[0] = x0 - r
    rate_ref[0] = rate


def _vq_group(x, cb0, cb1, cb2):
    m, T, d = x.shape
    grid = (m,)
    q, rate = pl.pallas_call(
        _vq_group_body,
        grid=grid,
        in_specs=[
            pl.BlockSpec((1, T, d), lambda i: (i, 0, 0)),
            pl.BlockSpec((1, cb0.shape[1], d), lambda i: (i, 0, 0)),
            pl.BlockSpec((1, cb1.shape[1], d), lambda i: (i, 0, 0)),
            pl.BlockSpec((1, cb2.shape[1], d), lambda i: (i, 0, 0)),
        ],
        out_specs=[
            pl.BlockSpec((1, T, d), lambda i: (i, 0, 0)),
            pl.BlockSpec((1, 1, 1), lambda i: (i, 0, 0)),
        ],
        out_shape=[
            jax.ShapeDtypeStruct((m, T, d), jnp.float32),
            jax.ShapeDtypeStruct((m, 1, 1), jnp.float32),
        ],
    )(x, cb0, cb1, cb2)
    return q, jnp.sum(rate) / jnp.float32(T * m)


def _to_tokens(x, m):
    B, c, H, W = x.shape
    d = c // m
    return x.reshape(B, m, d, H * W).transpose(1, 0, 3, 2).reshape(m, B * H * W, d)


def _from_tokens(q, B, c, H, W, m):
    d = c // m
    return q.reshape(m, B, H * W, d).transpose(1, 0, 3, 2).reshape(B, c, H, W)


def kernel(y0, y1, y2, y3, y4,
           cb0_0, cb0_1, cb0_2,
           cb1_0, cb1_1, cb1_2,
           cb2_0, cb2_1, cb2_2,
           cb3_0, cb3_1, cb3_2,
           cb4_0, cb4_1, cb4_2):
    ys = [y0, y1, y2, y3, y4]
    cbs = [[cb0_0, cb0_1, cb0_2],
           [cb1_0, cb1_1, cb1_2],
           [cb2_0, cb2_1, cb2_2],
           [cb3_0, cb3_1, cb3_2],
           [cb4_0, cb4_1, cb4_2]]
    B, _, H, W = y0.shape
    former = []
    dec = []
    rates = []
    f = None
    for i in range(5):
        m = cbs[i][0].shape[0]
        c = ys[i].shape[1]
        nin = ys[i] if f is None else ys[i] - f
        xt = _to_tokens(nin, m)
        qt, rate = _vq_group(xt, *cbs[i])
        q = _from_tokens(qt, B, c, H, W, m)
        dec.append(q if f is None else q + f)
        former.append(q)
        f = q if f is None else jnp.concatenate([f, q], axis=1)
        rates.append(rate)
    return jnp.concatenate(dec, axis=1), jnp.stack(rates)


# speedup vs baseline: 1.5424x; 1.3486x over previous
import jax
import jax.numpy as jnp
from jax.experimental import pallas as pl
from jax.experimental.pallas import tpu as pltpu

_EPS = 1e-07
_KT = 2048


def _vq_group_body(x_ref, cb0_ref, cb1_ref, cb2_ref,
                   cbt0_ref, cbt1_ref, cbt2_ref,
                   cbn0_ref, cbn1_ref, cbn2_ref,
                   q_ref, rate_ref):
    x0 = x_ref[0]
    T = x0.shape[1]
    r = x0
    rate = jnp.zeros((1, 1), jnp.float32)
    iota0 = jax.lax.broadcasted_iota(jnp.int32, (_KT, T), 0)
    for cb_ref, cbt_ref, cbn_ref in ((cb0_ref, cbt0_ref, cbn0_ref),
                                     (cb1_ref, cbt1_ref, cbn1_ref),
                                     (cb2_ref, cbt2_ref, cbn2_ref)):
        K = cb_ref.shape[1]
        KT = min(K, _KT)
        nt = K // KT
        io = iota0[:KT] if KT < _KT else iota0
        M = S = A = None
        for t in range(nt):
            cbt = cb_ref[0, t * KT:(t + 1) * KT, :]
            dots = jax.lax.dot_general(
                cbt, r, (((1,), (0,)), ((), ())),
                preferred_element_type=jnp.float32)
            tt = dots - cbn_ref[0, t * KT:(t + 1) * KT, :]
            tmax = jnp.max(tt, axis=0, keepdims=True)
            targ = jnp.min(jnp.where(tt == tmax, io, jnp.int32(K)),
                           axis=0, keepdims=True) + t * KT
            tsum = jnp.sum(jnp.exp(tt - tmax), axis=0, keepdims=True)
            if t == 0:
                M, S, A = tmax, tsum, targ
            else:
                better = tmax > M
                Mn = jnp.maximum(M, tmax)
                S = S * jnp.exp(M - Mn) + tsum * jnp.exp(tmax - Mn)
                A = jnp.where(better, targ, A)
                M = Mn
        sel = None
        for t in range(nt):
            cbt_t = cbt_ref[0, :, t * KT:(t + 1) * KT]
            oh = (io == (A - t * KT)).astype(jnp.float32)
            g = jax.lax.dot_general(
                cbt_t, oh, (((1,), (0,)), ((), ())),
                precision=jax.lax.Precision.HIGHEST,
                preferred_element_type=jnp.float32)
            sel = g if sel is None else sel + g
        r = r - sel
        rate = rate + jnp.sum(-jnp.log(1.0 / S + _EPS), keepdims=True)
    q_ref[0] = x0 - r
    rate_ref[0] = rate


def _vq_group(x, cb0, cb1, cb2):
    m, d, T = x.shape
    cbs2 = [cb0 * 2.0, cb1 * 2.0, cb2 * 2.0]
    cbts = [jnp.transpose(cb, (0, 2, 1)) for cb in (cb0, cb1, cb2)]
    cbns = [jnp.sum(cb * cb, axis=-1)[..., None] for cb in (cb0, cb1, cb2)]
    q, rate = pl.pallas_call(
        _vq_group_body,
        grid=(m,),
        in_specs=[
            pl.BlockSpec((1, d, T), lambda i: (i, 0, 0)),
            *[pl.BlockSpec((1, cb.shape[1], d), lambda i: (i, 0, 0))
              for cb in (cb0, cb1, cb2)],
            *[pl.BlockSpec((1, d, cb.shape[1]), lambda i: (i, 0, 0))
              for cb in (cb0, cb1, cb2)],
            *[pl.BlockSpec((1, cb.shape[1], 1), lambda i: (i, 0, 0))
              for cb in (cb0, cb1, cb2)],
        ],
        out_specs=[
            pl.BlockSpec((1, d, T), lambda i: (i, 0, 0)),
            pl.BlockSpec((1, 1, 1), lambda i: (i, 0, 0)),
        ],
        out_shape=[
            jax.ShapeDtypeStruct((m, d, T), jnp.float32),
            jax.ShapeDtypeStruct((m, 1, 1), jnp.float32),
        ],
        compiler_params=pltpu.CompilerParams(
            dimension_semantics=("parallel",)),
    )(x, *cbs2, *cbts, *cbns)
    return q, jnp.sum(rate) / jnp.float32(T * m)


def _to_tokens(x, m):
    B, c, H, W = x.shape
    d = c // m
    return x.reshape(B, m, d, H * W).transpose(1, 2, 0, 3).reshape(m, d, B * H * W)


def _from_tokens(q, B, c, H, W, m):
    d = c // m
    return q.reshape(m, d, B, H * W).transpose(2, 0, 1, 3).reshape(B, c, H, W)


def kernel(y0, y1, y2, y3, y4,
           cb0_0, cb0_1, cb0_2,
           cb1_0, cb1_1, cb1_2,
           cb2_0, cb2_1, cb2_2,
           cb3_0, cb3_1, cb3_2,
           cb4_0, cb4_1, cb4_2):
    ys = [y0, y1, y2, y3, y4]
    cbs = [[cb0_0, cb0_1, cb0_2],
           [cb1_0, cb1_1, cb1_2],
           [cb2_0, cb2_1, cb2_2],
           [cb3_0, cb3_1, cb3_2],
           [cb4_0, cb4_1, cb4_2]]
    B, _, H, W = y0.shape
    dec = []
    rates = []
    f = None
    for i in range(5):
        m = cbs[i][0].shape[0]
        c = ys[i].shape[1]
        nin = ys[i] if f is None else ys[i] - f
        xt = _to_tokens(nin, m)
        qt, rate = _vq_group(xt, *cbs[i])
        q = _from_tokens(qt, B, c, H, W, m)
        dec.append(q if f is None else q + f)
        f = q if f is None else jnp.concatenate([f, q], axis=1)
        rates.append(rate)
    return jnp.concatenate(dec, axis=1), jnp.stack(rates)


# speedup vs baseline: 1.5426x; 1.0001x over previous
import jax
import jax.numpy as jnp
from jax.experimental import pallas as pl
from jax.experimental.pallas import tpu as pltpu

_EPS = 1e-07
_KT = 2048


def _vq_group_body(x_ref, cb0_ref, cb1_ref, cb2_ref,
                   cbt0_ref, cbt1_ref, cbt2_ref,
                   cbn0_ref, cbn1_ref, cbn2_ref,
                   q_ref, rate_ref):
    x0 = x_ref[0]
    T = x0.shape[1]
    r = x0
    rate = jnp.zeros((1, 1), jnp.float32)
    iota0 = jax.lax.broadcasted_iota(jnp.int32, (_KT, T), 0)
    for cb_ref, cbt_ref, cbn_ref in ((cb0_ref, cbt0_ref, cbn0_ref),
                                     (cb1_ref, cbt1_ref, cbn1_ref),
                                     (cb2_ref, cbt2_ref, cbn2_ref)):
        K = cb_ref.shape[1]
        KT = min(K, _KT)
        nt = K // KT
        io = iota0[:KT] if KT < _KT else iota0
        M = S = A = None
        for t in range(nt):
            cbt = cb_ref[0, t * KT:(t + 1) * KT, :]
            dots = jax.lax.dot_general(
                cbt, r, (((1,), (0,)), ((), ())),
                preferred_element_type=jnp.float32)
            tt = dots - cbn_ref[0, t * KT:(t + 1) * KT, :]
            tmax = jnp.max(tt, axis=0, keepdims=True)
            targ = jnp.min(jnp.where(tt == tmax, io, jnp.int32(K)),
                           axis=0, keepdims=True) + t * KT
            tsum = jnp.sum(jnp.exp(tt - tmax), axis=0, keepdims=True)
            if t == 0:
                M, S, A = tmax, tsum, targ
            else:
                better = tmax > M
                Mn = jnp.maximum(M, tmax)
                S = S * jnp.exp(M - Mn) + tsum * jnp.exp(tmax - Mn)
                A = jnp.where(better, targ, A)
                M = Mn
        sel = None
        for t in range(nt):
            cbt_t = cbt_ref[0, :, t * KT:(t + 1) * KT]
            oh = (io == (A - t * KT)).astype(jnp.float32)
            g = jax.lax.dot_general(
                cbt_t, oh, (((1,), (0,)), ((), ())),
                precision=jax.lax.Precision.HIGHEST,
                preferred_element_type=jnp.float32)
            sel = g if sel is None else sel + g
        r = r - sel
        rate = rate + jnp.sum(-jnp.log(1.0 / S + _EPS), keepdims=True)
    q_ref[0] = x0 - r
    rate_ref[0] = rate


def _vq_group(x, cb0, cb1, cb2):
    m, d, T = x.shape
    cbs2 = [cb0 * 2.0, cb1 * 2.0, cb2 * 2.0]
    cbts = [jnp.transpose(cb, (0, 2, 1)) for cb in (cb0, cb1, cb2)]
    cbns = [jnp.sum(cb * cb, axis=-1)[..., None] for cb in (cb0, cb1, cb2)]
    q, rate = pl.pallas_call(
        _vq_group_body,
        grid=(m,),
        in_specs=[
            pl.BlockSpec((1, d, T), lambda i: (i, 0, 0)),
            *[pl.BlockSpec((1, cb.shape[1], d), lambda i: (i, 0, 0))
              for cb in (cb0, cb1, cb2)],
            *[pl.BlockSpec((1, d, cb.shape[1]), lambda i: (i, 0, 0))
              for cb in (cb0, cb1, cb2)],
            *[pl.BlockSpec((1, cb.shape[1], 1), lambda i: (i, 0, 0))
              for cb in (cb0, cb1, cb2)],
        ],
        out_specs=[
            pl.BlockSpec((1, d, T), lambda i: (i, 0, 0)),
            pl.BlockSpec((1, 1, 1), lambda i: (i, 0, 0)),
        ],
        out_shape=[
            jax.ShapeDtypeStruct((m, d, T), jnp.float32),
            jax.ShapeDtypeStruct((m, 1, 1), jnp.float32),
        ],
        compiler_params=pltpu.CompilerParams(
            dimension_semantics=("arbitrary",)),
    )(x, *cbs2, *cbts, *cbns)
    return q, jnp.sum(rate) / jnp.float32(T * m)


def _to_tokens(x, m):
    B, c, H, W = x.shape
    d = c // m
    return x.reshape(B, m, d, H * W).transpose(1, 2, 0, 3).reshape(m, d, B * H * W)


def _from_tokens(q, B, c, H, W, m):
    d = c // m
    return q.reshape(m, d, B, H * W).transpose(2, 0, 1, 3).reshape(B, c, H, W)


def kernel(y0, y1, y2, y3, y4,
           cb0_0, cb0_1, cb0_2,
           cb1_0, cb1_1, cb1_2,
           cb2_0, cb2_1, cb2_2,
           cb3_0, cb3_1, cb3_2,
           cb4_0, cb4_1, cb4_2):
    ys = [y0, y1, y2, y3, y4]
    cbs = [[cb0_0, cb0_1, cb0_2],
           [cb1_0, cb1_1, cb1_2],
           [cb2_0, cb2_1, cb2_2],
           [cb3_0, cb3_1, cb3_2],
           [cb4_0, cb4_1, cb4_2]]
    B, _, H, W = y0.shape
    dec = []
    rates = []
    f = None
    for i in range(5):
        m = cbs[i][0].shape[0]
        c = ys[i].shape[1]
        nin = ys[i] if f is None else ys[i] - f
        xt = _to_tokens(nin, m)
        qt, rate = _vq_group(xt, *cbs[i])
        q = _from_tokens(qt, B, c, H, W, m)
        dec.append(q if f is None else q + f)
        f = q if f is None else jnp.concatenate([f, q], axis=1)
        rates.append(rate)
    return jnp.concatenate(dec, axis=1), jnp.stack(rates)


# speedup vs baseline: 2.6032x; 1.6876x over previous
import jax
import jax.numpy as jnp
from jax.experimental import pallas as pl
from jax.experimental.pallas import tpu as pltpu

_EPS = 1e-07
_KT = 2048


def _vq_group_body(x_ref, cb0_ref, cb1_ref, cb2_ref,
                   cbt0_ref, cbt1_ref, cbt2_ref,
                   cbn0_ref, cbn1_ref, cbn2_ref,
                   q_ref, rate_ref):
    x0 = x_ref[0]
    T = x0.shape[1]
    r = x0
    rate = jnp.zeros((1, 1), jnp.float32)
    iota0 = jax.lax.broadcasted_iota(jnp.int32, (_KT, T), 0)
    for cb_ref, cbt_ref, cbn_ref in ((cb0_ref, cbt0_ref, cbn0_ref),
                                     (cb1_ref, cbt1_ref, cbn1_ref),
                                     (cb2_ref, cbt2_ref, cbn2_ref)):
        K = cb_ref.shape[1]
        KT = min(K, _KT)
        nt = K // KT
        io = iota0[:KT] if KT < _KT else iota0
        M = S = A = None
        for t in range(nt):
            cbt = cb_ref[0, t * KT:(t + 1) * KT, :]
            dots = jax.lax.dot_general(
                cbt, r, (((1,), (0,)), ((), ())),
                preferred_element_type=jnp.float32)
            tt = dots - cbn_ref[0, t * KT:(t + 1) * KT, :]
            tmax = jnp.max(tt, axis=0, keepdims=True)
            targ = jnp.min(jnp.where(tt == tmax, io, jnp.int32(K)),
                           axis=0, keepdims=True) + t * KT
            tsum = jnp.sum(jnp.exp(tt - tmax), axis=0, keepdims=True)
            if t == 0:
                M, S, A = tmax, tsum, targ
            else:
                better = tmax > M
                Mn = jnp.maximum(M, tmax)
                S = S * jnp.exp(M - Mn) + tsum * jnp.exp(tmax - Mn)
                A = jnp.where(better, targ, A)
                M = Mn
        sel = None
        for t in range(nt):
            c0 = cbt_ref[0, :, t * KT:(t + 1) * KT]
            hi = c0.astype(jnp.bfloat16)
            r1 = c0 - hi.astype(jnp.float32)
            mid = r1.astype(jnp.bfloat16)
            lo = (r1 - mid.astype(jnp.float32)).astype(jnp.bfloat16)
            oh = (io == (A - t * KT)).astype(jnp.bfloat16)
            g = None
            for part in (hi, mid, lo):
                gp = jax.lax.dot_general(
                    part, oh, (((1,), (0,)), ((), ())),
                    preferred_element_type=jnp.float32)
                g = gp if g is None else g + gp
            sel = g if sel is None else sel + g
        r = r - sel
        rate = rate + jnp.sum(-jnp.log(1.0 / S + _EPS), keepdims=True)
    q_ref[0] = x0 - r
    rate_ref[0] = rate


def _vq_group(x, cb0, cb1, cb2):
    m, d, T = x.shape
    cbs2 = [cb0 * 2.0, cb1 * 2.0, cb2 * 2.0]
    cbts = [jnp.transpose(cb, (0, 2, 1)) for cb in (cb0, cb1, cb2)]
    cbns = [jnp.sum(cb * cb, axis=-1)[..., None] for cb in (cb0, cb1, cb2)]
    q, rate = pl.pallas_call(
        _vq_group_body,
        grid=(m,),
        in_specs=[
            pl.BlockSpec((1, d, T), lambda i: (i, 0, 0)),
            *[pl.BlockSpec((1, cb.shape[1], d), lambda i: (i, 0, 0))
              for cb in (cb0, cb1, cb2)],
            *[pl.BlockSpec((1, d, cb.shape[1]), lambda i: (i, 0, 0))
              for cb in (cb0, cb1, cb2)],
            *[pl.BlockSpec((1, cb.shape[1], 1), lambda i: (i, 0, 0))
              for cb in (cb0, cb1, cb2)],
        ],
        out_specs=[
            pl.BlockSpec((1, d, T), lambda i: (i, 0, 0)),
            pl.BlockSpec((1, 1, 1), lambda i: (i, 0, 0)),
        ],
        out_shape=[
            jax.ShapeDtypeStruct((m, d, T), jnp.float32),
            jax.ShapeDtypeStruct((m, 1, 1), jnp.float32),
        ],
        compiler_params=pltpu.CompilerParams(
            dimension_semantics=("arbitrary",)),
    )(x, *cbs2, *cbts, *cbns)
    return q, jnp.sum(rate) / jnp.float32(T * m)


def _to_tokens(x, m):
    B, c, H, W = x.shape
    d = c // m
    return x.reshape(B, m, d, H * W).transpose(1, 2, 0, 3).reshape(m, d, B * H * W)


def _from_tokens(q, B, c, H, W, m):
    d = c // m
    return q.reshape(m, d, B, H * W).transpose(2, 0, 1, 3).reshape(B, c, H, W)


def kernel(y0, y1, y2, y3, y4,
           cb0_0, cb0_1, cb0_2,
           cb1_0, cb1_1, cb1_2,
           cb2_0, cb2_1, cb2_2,
           cb3_0, cb3_1, cb3_2,
           cb4_0, cb4_1, cb4_2):
    ys = [y0, y1, y2, y3, y4]
    cbs = [[cb0_0, cb0_1, cb0_2],
           [cb1_0, cb1_1, cb1_2],
           [cb2_0, cb2_1, cb2_2],
           [cb3_0, cb3_1, cb3_2],
           [cb4_0, cb4_1, cb4_2]]
    B, _, H, W = y0.shape
    dec = []
    rates = []
    f = None
    for i in range(5):
        m = cbs[i][0].shape[0]
        c = ys[i].shape[1]
        nin = ys[i] if f is None else ys[i] - f
        xt = _to_tokens(nin, m)
        qt, rate = _vq_group(xt, *cbs[i])
        q = _from_tokens(qt, B, c, H, W, m)
        dec.append(q if f is None else q + f)
        f = q if f is None else jnp.concatenate([f, q], axis=1)
        rates.append(rate)
    return jnp.concatenate(dec, axis=1), jnp.stack(rates)


# speedup vs baseline: 2.8617x; 1.0993x over previous
import jax
import jax.numpy as jnp
from jax.experimental import pallas as pl
from jax.experimental.pallas import tpu as pltpu

_EPS = 1e-07
_KT = 2048


def _vq_group_body(x_ref, cb0_ref, cb1_ref, cb2_ref,
                   cbt0_ref, cbt1_ref, cbt2_ref,
                   cbn0_ref, cbn1_ref, cbn2_ref,
                   q_ref, rate_ref):
    x0 = x_ref[0]
    T = x0.shape[1]
    r = x0
    rate = jnp.zeros((1, 1), jnp.float32)
    iota0 = jax.lax.broadcasted_iota(jnp.int32, (_KT, T), 0)
    for cb_ref, cbt_ref, cbn_ref in ((cb0_ref, cbt0_ref, cbn0_ref),
                                     (cb1_ref, cbt1_ref, cbn1_ref),
                                     (cb2_ref, cbt2_ref, cbn2_ref)):
        K = cb_ref.shape[1]
        KT = min(K, _KT)
        nt = K // KT
        io = iota0[:KT] if KT < _KT else iota0
        M = S = A = None
        r2 = r + r
        for t in range(nt):
            cbt = cb_ref[0, t * KT:(t + 1) * KT, :]
            dots = jax.lax.dot_general(
                cbt, r2, (((1,), (0,)), ((), ())),
                preferred_element_type=jnp.float32)
            tt = dots - cbn_ref[0, t * KT:(t + 1) * KT, :]
            tmax = jnp.max(tt, axis=0, keepdims=True)
            targ = jnp.min(jnp.where(tt == tmax, io, jnp.int32(K)),
                           axis=0, keepdims=True) + t * KT
            tsum = jnp.sum(jnp.exp(tt - tmax), axis=0, keepdims=True)
            if t == 0:
                M, S, A = tmax, tsum, targ
            else:
                better = tmax > M
                Mn = jnp.maximum(M, tmax)
                S = S * jnp.exp(M - Mn) + tsum * jnp.exp(tmax - Mn)
                A = jnp.where(better, targ, A)
                M = Mn
        sel = None
        for t in range(nt):
            c0 = cbt_ref[0, :, t * KT:(t + 1) * KT]
            hi = c0.astype(jnp.bfloat16)
            r1 = c0 - hi.astype(jnp.float32)
            mid = r1.astype(jnp.bfloat16)
            lo = (r1 - mid.astype(jnp.float32)).astype(jnp.bfloat16)
            oh = (io == (A - t * KT)).astype(jnp.bfloat16)
            g = None
            for part in (hi, mid, lo):
                gp = jax.lax.dot_general(
                    part, oh, (((1,), (0,)), ((), ())),
                    preferred_element_type=jnp.float32)
                g = gp if g is None else g + gp
            sel = g if sel is None else sel + g
        r = r - sel
        rate = rate + jnp.sum(-jnp.log(1.0 / S + _EPS), keepdims=True)
    q_ref[0] = x0 - r
    rate_ref[0] = rate


def _vq_group(x, cb0, cb1, cb2):
    m, d, T = x.shape
    cbts = [jnp.transpose(cb, (0, 2, 1)) for cb in (cb0, cb1, cb2)]
    cbns = [jnp.sum(cb * cb, axis=-1)[..., None] for cb in (cb0, cb1, cb2)]
    q, rate = pl.pallas_call(
        _vq_group_body,
        grid=(m,),
        in_specs=[
            pl.BlockSpec((1, d, T), lambda i: (i, 0, 0)),
            *[pl.BlockSpec((1, cb.shape[1], d), lambda i: (i, 0, 0))
              for cb in (cb0, cb1, cb2)],
            *[pl.BlockSpec((1, d, cb.shape[1]), lambda i: (i, 0, 0))
              for cb in (cb0, cb1, cb2)],
            *[pl.BlockSpec((1, cb.shape[1], 1), lambda i: (i, 0, 0))
              for cb in (cb0, cb1, cb2)],
        ],
        out_specs=[
            pl.BlockSpec((1, d, T), lambda i: (i, 0, 0)),
            pl.BlockSpec((1, 1, 1), lambda i: (i, 0, 0)),
        ],
        out_shape=[
            jax.ShapeDtypeStruct((m, d, T), jnp.float32),
            jax.ShapeDtypeStruct((m, 1, 1), jnp.float32),
        ],
        compiler_params=pltpu.CompilerParams(
            dimension_semantics=("arbitrary",)),
    )(x, cb0, cb1, cb2, *cbts, *cbns)
    return q, jnp.sum(rate) / jnp.float32(T * m)


def _to_tokens(x, m):
    B, c, H, W = x.shape
    d = c // m
    return x.reshape(B, m, d, H * W).transpose(1, 2, 0, 3).reshape(m, d, B * H * W)


def _from_tokens(q, B, c, H, W, m):
    d = c // m
    return q.reshape(m, d, B, H * W).transpose(2, 0, 1, 3).reshape(B, c, H, W)


def kernel(y0, y1, y2, y3, y4,
           cb0_0, cb0_1, cb0_2,
           cb1_0, cb1_1, cb1_2,
           cb2_0, cb2_1, cb2_2,
           cb3_0, cb3_1, cb3_2,
           cb4_0, cb4_1, cb4_2):
    ys = [y0, y1, y2, y3, y4]
    cbs = [[cb0_0, cb0_1, cb0_2],
           [cb1_0, cb1_1, cb1_2],
           [cb2_0, cb2_1, cb2_2],
           [cb3_0, cb3_1, cb3_2],
           [cb4_0, cb4_1, cb4_2]]
    B, _, H, W = y0.shape
    dec = []
    rates = []
    f = None
    for i in range(5):
        m = cbs[i][0].shape[0]
        c = ys[i].shape[1]
        nin = ys[i] if f is None else ys[i] - f
        xt = _to_tokens(nin, m)
        qt, rate = _vq_group(xt, *cbs[i])
        q = _from_tokens(qt, B, c, H, W, m)
        dec.append(q if f is None else q + f)
        f = q if f is None else jnp.concatenate([f, q], axis=1)
        rates.append(rate)
    return jnp.concatenate(dec, axis=1), jnp.stack(rates)
